# baseline (speedup 1.0000x reference)
import functools

import jax
import jax.numpy as jnp
from jax import lax
from jax.experimental import pallas as pl
from jax.experimental.pallas import tpu as pltpu
from jax.experimental.pallas import tpu_sc as plsc

BATCH = 16384
D = 64
L = 16
CW = 256
NCHUNK = 999936 // CW
TAIL0 = NCHUNK * CW
STAGE = 128
DUMMY = BATCH
NVREG = BATCH // L


@functools.cache
def _build_sc():
    info = plsc.get_sparse_core_info()
    NC = info.num_cores
    NW = NC * info.num_subcores
    base_nt = NCHUNK // NW
    rem = NCHUNK - base_nt * NW
    mesh = plsc.VectorSubcoreMesh(core_axis_name="c", subcore_axis_name="s")

    @functools.partial(
        pl.kernel,
        mesh=mesh,
        out_type=(jax.ShapeDtypeStruct((BATCH + L, 128), jnp.float32),
                  jax.ShapeDtypeStruct((BATCH + L, 128), jnp.float32)),
        compiler_params=pltpu.CompilerParams(
            needs_layout_passes=False, use_tc_tiling_on_sc=True),
        scratch_types=(
            [pltpu.VMEM((BATCH,), jnp.int32)]
            + [pltpu.VMEM((BATCH + L,), jnp.int32)] * 2
            + [pltpu.VMEM((D, CW), jnp.float32)] * 2
            + [pltpu.VMEM((STAGE, 128), jnp.float32)]
            + [pltpu.VMEM((STAGE,), jnp.int32)]
            + [pltpu.SemaphoreType.DMA] * 3
        ),
    )
    def scan(u_ids_hbm, i_ids_hbm, ut_hbm, it_hbm, ue_hbm, ie_hbm,
             idsv, listb, listid, buf0, buf1, stage, bidx, sem0, sem1, semf):
        bufs = (buf0, buf1)
        sems = (sem0, sem1)
        wid = lax.axis_index("s") * NC + lax.axis_index("c")
        nt = base_nt + jnp.where(wid < rem, 1, 0)
        iota = lax.iota(jnp.int32, L)
        lane0 = iota == 0
        rowvecs = [j * L + iota for j in range(4)]

        def reset_bidx():
            for j in range(STAGE // L):
                bidx[pl.ds(j * L, L)] = jnp.full((L,), DUMMY, jnp.int32)

        def one_table(ids_hbm, tab, out_hbm):
            pltpu.sync_copy(ids_hbm, idsv)

            def sel(i, cnt):
                v = idsv[pl.ds(i * L, L)]
                m = jnp.bitwise_and(
                    lax.shift_right_logical(v, 8), NW - 1) == wid
                plsc.store_compressed(listb.at[pl.ds(cnt, L)], i * L + iota, mask=m)
                plsc.store_compressed(listid.at[pl.ds(cnt, L)], v, mask=m)
                return cnt + jnp.sum(m.astype(jnp.int32))

            cnt = lax.fori_loop(0, NVREG, sel, jnp.int32(0))
            nq = (cnt + (L - 1)) // L
            reset_bidx()

            def fire(t, s):
                col0 = pl.multiple_of((t * NW + wid) * CW, 128)
                pltpu.async_copy(tab.at[:, pl.ds(col0, CW)], bufs[s], sems[s])

            def drain(s):
                pltpu.make_async_copy(
                    tab.at[:, pl.ds(0, CW)], bufs[s], sems[s]).wait()

            def flush():
                pltpu.async_copy(stage, out_hbm.at[bidx], semf).wait()
                reset_bidx()

            @pl.when(nt > 0)
            def _():
                fire(jnp.int32(0), 0)

            @pl.when(nt > 1)
            def _():
                fire(jnp.int32(1), 1)

            def round_body(r, nstage):
                for s in range(2):
                    t = r * 2 + s
                    in_range = t < nt

                    def scanq(q, nst):
                        vb = listb[pl.ds(q * L, L)]
                        vid = listid[pl.ds(q * L, L)]
                        valid = (q * L + iota) < cnt
                        m = (lax.shift_right_logical(vid, 8)
                             == (t * NW + wid)) & valid
                        nm = jnp.sum(m.astype(jnp.int32))

                        def have(nst2):
                            mb = jnp.where(m, vb, 0)
                            mj = jnp.where(m, jnp.bitwise_and(
                                vid, CW - 1), 0)
                            plsc.store_compressed(
                                listb.at[pl.ds(BATCH, L)], mb, mask=m)
                            plsc.store_compressed(
                                listid.at[pl.ds(BATCH, L)], mj, mask=m)
                            cb = listb[pl.ds(BATCH, L)]
                            cj = listid[pl.ds(BATCH, L)]
                            res = nst2
                            for e in range(L):
                                active = e < nm
                                b_e = cb[e]
                                j_e = cj[e]
                                spos = jnp.bitwise_and(res, STAGE - 1)
                                colv = jnp.full((L,), j_e, jnp.int32)
                                rowv = jnp.full((L,), spos, jnp.int32)

                                @pl.when(active)
                                def _():
                                    for g in range(4):
                                        v = plsc.load_gather(
                                            bufs[s], [rowvecs[g], colv])
                                        plsc.store_scatter(
                                            stage,
                                            [rowv, g * L + iota], v)
                                    plsc.store_scatter(
                                        bidx,
                                        [jnp.full((L,), spos, jnp.int32)],
                                        jnp.full((L,), b_e, jnp.int32),
                                        mask=lane0)

                                res = res + active.astype(jnp.int32)

                                @pl.when(active & (jnp.bitwise_and(
                                    res, STAGE - 1) == 0))
                                def _():
                                    flush()
                            return res

                        return lax.cond(nm > 0, have, lambda x: x, nst)

                    def do_chunk(nst):
                        drain(s)
                        out = lax.fori_loop(0, nq, scanq, nst)

                        @pl.when(t + 2 < nt)
                        def _():
                            fire(t + 2, s)

                        return out

                    nstage = lax.cond(in_range, do_chunk, lambda x: x, nstage)
                return nstage

            nrounds = (base_nt + 1 + 1) // 2
            lax.fori_loop(0, nrounds, round_body, jnp.int32(0))
            flush()

        one_table(u_ids_hbm, ut_hbm, ue_hbm)
        one_table(i_ids_hbm, it_hbm, ie_hbm)

    return scan


@functools.cache
def _build_tc():
    BLK = 2048
    grid = BATCH // BLK

    def body(ue_ref, ie_ref, uid_ref, iid_ref, utail_ref, itail_ref, out_ref):
        uid = uid_ref[...]
        iid = iid_ref[...]
        io64 = lax.broadcasted_iota(jnp.int32, (BLK, D), 1)

        def patch(rows, ids, tail_ref):
            flag = ids >= TAIL0
            oh = (io64 == (ids - TAIL0)).astype(jnp.float32)
            trows = jax.lax.dot_general(
                oh, tail_ref[...], (((1,), (0,)), ((), ())),
                precision=jax.lax.Precision.HIGHEST,
                preferred_element_type=jnp.float32)
            return jnp.where(flag, trows, rows)

        ue = patch(ue_ref[:, :D], uid, utail_ref)
        ie = patch(ie_ref[:, :D], iid, itail_ref)
        out_ref[...] = jnp.sum(ue * ie, axis=1)

    return pl.pallas_call(
        body,
        grid=(grid,),
        in_specs=[
            pl.BlockSpec((BLK, 128), lambda i: (i, 0)),
            pl.BlockSpec((BLK, 128), lambda i: (i, 0)),
            pl.BlockSpec((BLK, 1), lambda i: (i, 0)),
            pl.BlockSpec((BLK, 1), lambda i: (i, 0)),
            pl.BlockSpec((D, D), lambda i: (0, 0)),
            pl.BlockSpec((D, D), lambda i: (0, 0)),
        ],
        out_specs=pl.BlockSpec((BLK,), lambda i: (i,)),
        out_shape=jax.ShapeDtypeStruct((BATCH,), jnp.float32),
    )


def kernel(u_ids, i_ids, user_table, item_table):
    uid = u_ids.astype(jnp.int32)
    iid = i_ids.astype(jnp.int32)
    ue, ie = _build_sc()(uid, iid, user_table.T, item_table.T)
    return _build_tc()(ue, ie, uid[:, None], iid[:, None],
                       user_table[TAIL0:], item_table[TAIL0:])


# speedup vs baseline: 7.1119x; 7.1119x over previous
import functools

import jax
import jax.numpy as jnp
from jax import lax
from jax.experimental import pallas as pl
from jax.experimental.pallas import tpu as pltpu
from jax.experimental.pallas import tpu_sc as plsc

BATCH = 16384
D = 64
L = 16
NBUF = 16
H = 256
SG = 16


@functools.cache
def _build():
    info = plsc.get_sparse_core_info()
    NC = info.num_cores
    NW = NC * info.num_subcores
    b_per_w = BATCH // NW
    n_half = b_per_w // H
    nsg = H // SG
    ng = H // L
    mesh = plsc.VectorSubcoreMesh(core_axis_name="c", subcore_axis_name="s")

    @functools.partial(
        pl.kernel,
        mesh=mesh,
        out_type=jax.ShapeDtypeStruct((BATCH,), jnp.float32),
        compiler_params=pltpu.CompilerParams(
            needs_layout_passes=False, use_tc_tiling_on_sc=True),
        scratch_types=(
            [pltpu.VMEM((b_per_w,), jnp.int32)] * 2
            + [pltpu.VMEM((32, 128), jnp.float32)] * NBUF
            + [pltpu.VMEM((H * D,), jnp.float32)] * 2
            + [pltpu.VMEM((b_per_w,), jnp.float32)]
            + [pltpu.SemaphoreType.DMA] * NBUF
        ),
    )
    def bprmf(u_ids_hbm, i_ids_hbm, ut_hbm, it_hbm, out_hbm,
              uidv, iidv, *rest):
        bufs = rest[:NBUF]
        urows, irows, outv = rest[NBUF:NBUF + 3]
        sems = rest[NBUF + 3:]

        wid = lax.axis_index("s") * NC + lax.axis_index("c")
        base = wid * b_per_w
        pltpu.sync_copy(u_ids_hbm.at[pl.ds(base, b_per_w)], uidv)
        pltpu.sync_copy(i_ids_hbm.at[pl.ds(base, b_per_w)], iidv)

        iota = lax.iota(jnp.int32, L)
        rowvecs = [j * L + iota for j in range(2)]

        def fire(tab, slot, uid, h):
            col0 = pl.multiple_of(jnp.bitwise_and(uid, -128), 128)
            pltpu.async_copy(
                tab.at[pl.ds(h * 32, 32), pl.ds(col0, 128)],
                bufs[slot], sems[slot])

        def drain(tab, slot):
            pltpu.make_async_copy(
                tab.at[pl.ds(0, 32), pl.ds(0, 128)],
                bufs[slot], sems[slot]).wait()

        def extract(slot, uid, rows, bglobal, h):
            col = jnp.full((L,), jnp.bitwise_and(uid, 127), jnp.int32)
            for j in range(2):
                v = plsc.load_gather(bufs[slot], [rowvecs[j], col])
                rows[pl.ds(bglobal * D + h * 32 + j * L, L)] = v

        def fetch_phase(tab, idv, rows, half):
            off = half * H
            first = idv[pl.ds(off, L)]
            for k in range(NBUF):
                fire(tab, k, first[k // 2], k & 1)

            def group(g, idvec):
                nxt = idv[pl.ds(off + jnp.minimum((g + 1) * SG, H - SG), L)]
                for k in range(2 * SG):
                    slot = k % NBUF
                    drain(tab, slot)
                    extract(slot, idvec[k // 2], rows, g * SG + k // 2, k & 1)
                    if k < NBUF:
                        fire(tab, slot, idvec[k // 2 + 8], k & 1)
                    else:
                        @pl.when(g < nsg - 1)
                        def _():
                            fire(tab, slot, nxt[(k - NBUF) // 2], k & 1)
                return nxt

            lax.fori_loop(0, nsg, group, first)

        def dot_phase(half):
            def group(g, carry):
                rbase = (g * L + iota) * D
                acc = jnp.zeros((L,), jnp.float32)
                for d in range(D):
                    u = plsc.load_gather(urows, [rbase + d])
                    v = plsc.load_gather(irows, [rbase + d])
                    acc = acc + u * v
                outv[pl.ds(half * H + g * L, L)] = acc
                return carry

            lax.fori_loop(0, ng, group, 0)

        for half in range(n_half):
            fetch_phase(ut_hbm, uidv, urows, half)
            fetch_phase(it_hbm, iidv, irows, half)
            dot_phase(half)

        pltpu.sync_copy(outv, out_hbm.at[pl.ds(base, b_per_w)])

    return bprmf


def kernel(u_ids, i_ids, user_table, item_table):
    return _build()(u_ids.astype(jnp.int32), i_ids.astype(jnp.int32),
                    user_table.T, item_table.T)
